# baseline (speedup 1.0000x reference)
import jax
import jax.numpy as jnp
from jax import lax
from jax.experimental import pallas as pl
from jax.experimental.pallas import tpu as pltpu
from jax.experimental.pallas import tpu_sc as plsc

EMBED_DIM = 256
N_EMBED = 8192
DECAY = 0.99
EPS = 1e-05

S = 8 * 576
CODE_BLK = 1024
N_CODE_BLKS = N_EMBED // CODE_BLK
COMBO = 2 * EMBED_DIM + 128

NC, NS = 2, 16
HROWS = S // (NC * NS * 2)


def _argmin_body(samples_ref, mean_ref, csize_ref,
                 idx_ref, sumcs_ref, best_ref, barg_ref, s2_ref):
    j = pl.program_id(0)
    samples = samples_ref[...]

    @pl.when(j == 0)
    def _init():
        s2_ref[...] = jnp.sum(samples * samples, axis=1, keepdims=True)
        best_ref[...] = jnp.full((S, 1), jnp.inf, jnp.float32)
        barg_ref[...] = jnp.zeros((S, 1), jnp.int32)
        sumcs_ref[...] = jnp.sum(csize_ref[...], keepdims=True).reshape(1, 1)

    mean = mean_ref[...]
    mm = jnp.dot(samples, mean, preferred_element_type=jnp.float32)
    m2 = jnp.sum(mean * mean, axis=0, keepdims=True)
    dist = s2_ref[...] - 2.0 * mm + m2

    local_min = jnp.min(dist, axis=1, keepdims=True)
    col = jax.lax.broadcasted_iota(jnp.int32, dist.shape, 1)
    local_arg = jnp.min(
        jnp.where(dist == local_min, col, jnp.int32(2**30)),
        axis=1, keepdims=True) + j * CODE_BLK

    better = local_min < best_ref[...]
    best_ref[...] = jnp.where(better, local_min, best_ref[...])
    barg_ref[...] = jnp.where(better, local_arg, barg_ref[...])

    @pl.when(j == N_CODE_BLKS - 1)
    def _done():
        idx_ref[...] = barg_ref[...]


def _stats_body(samples_ref, idx_ref, csum_ref, csize_ref, ns_ref, tab_ref):
    j = pl.program_id(0)
    idx = idx_ref[...]
    col = jax.lax.broadcasted_iota(jnp.int32, (S, CODE_BLK), 1) + j * CODE_BLK
    onehot = (idx == col).astype(jnp.float32)

    counts = jnp.sum(onehot, axis=0, keepdims=True)
    bsum = jax.lax.dot_general(
        samples_ref[...], onehot, (((0,), (0,)), ((), ())),
        preferred_element_type=jnp.float32)

    csz = csize_ref[:, pl.ds(j * CODE_BLK, CODE_BLK)]
    ns = ns_ref[0, 0]
    usize = csz * DECAY + counts * (1.0 - DECAY)
    sm = (usize + EPS) * ns / (ns + N_EMBED * EPS)
    inv = 1.0 / sm

    tab_ref[:, pl.ds(0, EMBED_DIM)] = bsum.T
    tab_ref[:, pl.ds(EMBED_DIM, EMBED_DIM)] = csum_ref[...].T
    tab_ref[:, pl.ds(2 * EMBED_DIM, 128)] = jnp.broadcast_to(
        inv.T, (CODE_BLK, 128))


def _sc_body(samples_h, idx_h, tab_h, out_h, loss_h,
             idxg, gbuf, sbuf, lossv, sem):
    c = lax.axis_index("c")
    s = lax.axis_index("s")
    wid = s * NC + c

    lacc0 = jnp.zeros((16,), jnp.float32)

    for h in range(2):
        base = wid * (2 * HROWS) + h * HROWS
        pltpu.sync_copy(idx_h.at[pl.ds(base, HROWS)], idxg)
        cp = pltpu.async_copy(tab_h.at[idxg], gbuf, sem)
        pltpu.sync_copy(samples_h.at[pl.ds(base, HROWS)], sbuf)
        cp.wait()

        def _row(r, lacc):
            inv = gbuf[r, pl.ds(2 * EMBED_DIM, 16)]
            ca = DECAY * inv
            cb = (1.0 - DECAY) * inv
            for seg in range(16):
                sl = pl.ds(seg * 16, 16)
                q = (ca * gbuf[r, pl.ds(EMBED_DIM + seg * 16, 16)]
                     + cb * gbuf[r, pl.ds(seg * 16, 16)])
                d = sbuf[r, sl] - q
                lacc = lacc + d * d
                sbuf[r, sl] = q
            return lacc

        lacc0 = lax.fori_loop(0, HROWS, _row, lacc0)
        pltpu.sync_copy(sbuf, out_h.at[pl.ds(base, HROWS)])

    lossv[...] = lacc0
    pltpu.sync_copy(lossv, loss_h.at[pl.ds(wid * 16, 16)])


@jax.jit
def kernel(inputs, cluster_mean, cluster_size, cluster_sum):
    samples = jnp.reshape(inputs, (S, EMBED_DIM))
    csize_2d = jnp.reshape(cluster_size, (1, N_EMBED))

    idx2d, sumcs = pl.pallas_call(
        _argmin_body,
        grid=(N_CODE_BLKS,),
        in_specs=[
            pl.BlockSpec((S, EMBED_DIM), lambda j: (0, 0)),
            pl.BlockSpec((EMBED_DIM, CODE_BLK), lambda j: (0, j)),
            pl.BlockSpec((1, N_EMBED), lambda j: (0, 0)),
        ],
        out_specs=[
            pl.BlockSpec((S, 1), lambda j: (0, 0)),
            pl.BlockSpec((1, 1), lambda j: (0, 0)),
        ],
        out_shape=[
            jax.ShapeDtypeStruct((S, 1), jnp.int32),
            jax.ShapeDtypeStruct((1, 1), jnp.float32),
        ],
        scratch_shapes=[
            pltpu.VMEM((S, 1), jnp.float32),
            pltpu.VMEM((S, 1), jnp.int32),
            pltpu.VMEM((S, 1), jnp.float32),
        ],
    )(samples, cluster_mean, csize_2d)

    n_sample = sumcs * DECAY + (1.0 - DECAY) * float(S)

    tab = pl.pallas_call(
        _stats_body,
        grid=(N_CODE_BLKS,),
        in_specs=[
            pl.BlockSpec((S, EMBED_DIM), lambda j: (0, 0)),
            pl.BlockSpec((S, 1), lambda j: (0, 0)),
            pl.BlockSpec((EMBED_DIM, CODE_BLK), lambda j: (0, j)),
            pl.BlockSpec((1, N_EMBED), lambda j: (0, 0)),
            pl.BlockSpec((1, 1), lambda j: (0, 0)),
        ],
        out_specs=pl.BlockSpec((CODE_BLK, COMBO), lambda j: (j, 0)),
        out_shape=jax.ShapeDtypeStruct((N_EMBED, COMBO), jnp.float32),
    )(samples, idx2d, cluster_sum, csize_2d, n_sample)

    idx = jnp.reshape(idx2d, (S,))

    sc = pl.kernel(
        _sc_body,
        out_type=[
            jax.ShapeDtypeStruct((S, EMBED_DIM), jnp.float32),
            jax.ShapeDtypeStruct((NC * NS * 16,), jnp.float32),
        ],
        mesh=plsc.VectorSubcoreMesh(core_axis_name="c", subcore_axis_name="s",
                                    num_cores=NC, num_subcores=NS),
        scratch_types=[
            pltpu.VMEM((HROWS,), jnp.int32),
            pltpu.VMEM((HROWS, COMBO), jnp.float32),
            pltpu.VMEM((HROWS, EMBED_DIM), jnp.float32),
            pltpu.VMEM((16,), jnp.float32),
            pltpu.SemaphoreType.DMA,
        ],
    )
    out, loss_parts = sc(samples, idx, tab)

    outputs = jnp.reshape(out, inputs.shape)
    e_loss = jnp.sum(loss_parts) / float(S * EMBED_DIM)
    return (outputs, 0.25 * e_loss)


# speedup vs baseline: 1.0566x; 1.0566x over previous
import jax
import jax.numpy as jnp
from jax import lax
from jax.experimental import pallas as pl
from jax.experimental.pallas import tpu as pltpu
from jax.experimental.pallas import tpu_sc as plsc

EMBED_DIM = 256
N_EMBED = 8192
DECAY = 0.99
EPS = 1e-05

S = 8 * 576
CODE_BLK = 1024
N_CODE_BLKS = N_EMBED // CODE_BLK
COMBO = 2 * EMBED_DIM + 128

NC, NS = 2, 16
HROWS = S // (NC * NS * 2)


def _argmin_body(samples_ref, mean_ref, csize_ref,
                 idx_ref, sumcs_ref, best_ref, barg_ref, s2_ref, sneg_ref,
                 colf_ref):
    j = pl.program_id(0)

    @pl.when(j == 0)
    def _init():
        samples = samples_ref[...]
        sneg_ref[...] = samples * (-2.0)
        s2_ref[...] = jnp.sum(samples * samples, axis=1, keepdims=True)
        best_ref[...] = jnp.full((S, 1), jnp.inf, jnp.float32)
        barg_ref[...] = jnp.zeros((S, 1), jnp.int32)
        sumcs_ref[...] = jnp.sum(csize_ref[...], keepdims=True).reshape(1, 1)
        colf_ref[...] = jax.lax.broadcasted_iota(
            jnp.int32, (S, CODE_BLK), 1).astype(jnp.float32)

    mean = mean_ref[...]
    mm2 = jnp.dot(sneg_ref[...], mean, preferred_element_type=jnp.float32)
    m2 = jnp.sum(mean * mean, axis=0, keepdims=True)
    dist = (s2_ref[...] + mm2) + m2

    local_min = jnp.min(dist, axis=1, keepdims=True)
    local_argf = jnp.min(
        jnp.where(dist == local_min, colf_ref[...], jnp.float32(1e9)),
        axis=1, keepdims=True)
    local_arg = local_argf.astype(jnp.int32) + j * CODE_BLK

    better = local_min < best_ref[...]
    best_ref[...] = jnp.where(better, local_min, best_ref[...])
    barg_ref[...] = jnp.where(better, local_arg, barg_ref[...])

    @pl.when(j == N_CODE_BLKS - 1)
    def _done():
        idx_ref[...] = barg_ref[...]


def _stats_body(samples_ref, idx_ref, csum_ref, csize_ref, ns_ref, tab_ref):
    j = pl.program_id(0)
    idx = idx_ref[...]
    col = jax.lax.broadcasted_iota(jnp.int32, (S, CODE_BLK), 1) + j * CODE_BLK
    onehot = (idx == col).astype(jnp.float32)

    counts = jnp.sum(onehot, axis=0, keepdims=True)
    bsum = jax.lax.dot_general(
        samples_ref[...], onehot, (((0,), (0,)), ((), ())),
        preferred_element_type=jnp.float32)

    csz = csize_ref[:, pl.ds(j * CODE_BLK, CODE_BLK)]
    ns = ns_ref[0, 0]
    usize = csz * DECAY + counts * (1.0 - DECAY)
    sm = (usize + EPS) * ns / (ns + N_EMBED * EPS)
    inv = 1.0 / sm

    tab_ref[:, pl.ds(0, EMBED_DIM)] = bsum.T
    tab_ref[:, pl.ds(EMBED_DIM, EMBED_DIM)] = csum_ref[...].T
    tab_ref[:, pl.ds(2 * EMBED_DIM, 128)] = jnp.broadcast_to(
        inv.T, (CODE_BLK, 128))


def _sc_body(samples_h, idx_h, tab_h, out_h, loss_h,
             idxg, gbuf, sbuf, lossv, sem):
    c = lax.axis_index("c")
    s = lax.axis_index("s")
    wid = s * NC + c

    lacc0 = jnp.zeros((16,), jnp.float32)

    for h in range(2):
        base = wid * (2 * HROWS) + h * HROWS
        pltpu.sync_copy(idx_h.at[pl.ds(base, HROWS)], idxg)
        cp = pltpu.async_copy(tab_h.at[idxg], gbuf, sem)
        pltpu.sync_copy(samples_h.at[pl.ds(base, HROWS)], sbuf)
        cp.wait()

        def _row(r, lacc):
            inv = gbuf[r, pl.ds(2 * EMBED_DIM, 16)]
            ca = DECAY * inv
            cb = (1.0 - DECAY) * inv
            for seg in range(16):
                sl = pl.ds(seg * 16, 16)
                q = (ca * gbuf[r, pl.ds(EMBED_DIM + seg * 16, 16)]
                     + cb * gbuf[r, pl.ds(seg * 16, 16)])
                d = sbuf[r, sl] - q
                lacc = lacc + d * d
                sbuf[r, sl] = q
            return lacc

        lacc0 = lax.fori_loop(0, HROWS, _row, lacc0)
        pltpu.sync_copy(sbuf, out_h.at[pl.ds(base, HROWS)])

    lossv[...] = lacc0
    pltpu.sync_copy(lossv, loss_h.at[pl.ds(wid * 16, 16)])


@jax.jit
def kernel(inputs, cluster_mean, cluster_size, cluster_sum):
    samples = jnp.reshape(inputs, (S, EMBED_DIM))
    csize_2d = jnp.reshape(cluster_size, (1, N_EMBED))

    idx2d, sumcs = pl.pallas_call(
        _argmin_body,
        grid=(N_CODE_BLKS,),
        in_specs=[
            pl.BlockSpec((S, EMBED_DIM), lambda j: (0, 0)),
            pl.BlockSpec((EMBED_DIM, CODE_BLK), lambda j: (0, j)),
            pl.BlockSpec((1, N_EMBED), lambda j: (0, 0)),
        ],
        out_specs=[
            pl.BlockSpec((S, 1), lambda j: (0, 0)),
            pl.BlockSpec((1, 1), lambda j: (0, 0)),
        ],
        out_shape=[
            jax.ShapeDtypeStruct((S, 1), jnp.int32),
            jax.ShapeDtypeStruct((1, 1), jnp.float32),
        ],
        scratch_shapes=[
            pltpu.VMEM((S, 1), jnp.float32),
            pltpu.VMEM((S, 1), jnp.int32),
            pltpu.VMEM((S, 1), jnp.float32),
            pltpu.VMEM((S, EMBED_DIM), jnp.float32),
            pltpu.VMEM((S, CODE_BLK), jnp.float32),
        ],
    )(samples, cluster_mean, csize_2d)

    n_sample = sumcs * DECAY + (1.0 - DECAY) * float(S)

    tab = pl.pallas_call(
        _stats_body,
        grid=(N_CODE_BLKS,),
        in_specs=[
            pl.BlockSpec((S, EMBED_DIM), lambda j: (0, 0)),
            pl.BlockSpec((S, 1), lambda j: (0, 0)),
            pl.BlockSpec((EMBED_DIM, CODE_BLK), lambda j: (0, j)),
            pl.BlockSpec((1, N_EMBED), lambda j: (0, 0)),
            pl.BlockSpec((1, 1), lambda j: (0, 0)),
        ],
        out_specs=pl.BlockSpec((CODE_BLK, COMBO), lambda j: (j, 0)),
        out_shape=jax.ShapeDtypeStruct((N_EMBED, COMBO), jnp.float32),
    )(samples, idx2d, cluster_sum, csize_2d, n_sample)

    idx = jnp.reshape(idx2d, (S,))

    sc = pl.kernel(
        _sc_body,
        out_type=[
            jax.ShapeDtypeStruct((S, EMBED_DIM), jnp.float32),
            jax.ShapeDtypeStruct((NC * NS * 16,), jnp.float32),
        ],
        mesh=plsc.VectorSubcoreMesh(core_axis_name="c", subcore_axis_name="s",
                                    num_cores=NC, num_subcores=NS),
        scratch_types=[
            pltpu.VMEM((HROWS,), jnp.int32),
            pltpu.VMEM((HROWS, COMBO), jnp.float32),
            pltpu.VMEM((HROWS, EMBED_DIM), jnp.float32),
            pltpu.VMEM((16,), jnp.float32),
            pltpu.SemaphoreType.DMA,
        ],
    )
    out, loss_parts = sc(samples, idx, tab)

    outputs = jnp.reshape(out, inputs.shape)
    e_loss = jnp.sum(loss_parts) / float(S * EMBED_DIM)
    return (outputs, 0.25 * e_loss)


# speedup vs baseline: 1.1769x; 1.1139x over previous
import jax
import jax.numpy as jnp
from jax import lax
from jax.experimental import pallas as pl
from jax.experimental.pallas import tpu as pltpu
from jax.experimental.pallas import tpu_sc as plsc

EMBED_DIM = 256
N_EMBED = 8192
DECAY = 0.99
EPS = 1e-05

S = 8 * 576
CODE_BLK = 1024
N_CODE_BLKS = N_EMBED // CODE_BLK
COMBO = 2 * EMBED_DIM + 128

NC, NS = 2, 16
HROWS = S // (NC * NS * 2)


def _argmin_body(samples_ref, mean_ref, csize_ref,
                 idx_ref, sumcs_ref, best_ref, barg_ref, s2_ref, sneg_ref,
                 colf_ref):
    j = pl.program_id(0)

    @pl.when(j == 0)
    def _init():
        samples = samples_ref[...]
        sneg_ref[...] = samples * (-2.0)
        s2_ref[...] = jnp.sum(samples * samples, axis=1, keepdims=True)
        best_ref[...] = jnp.full((S, 1), jnp.inf, jnp.float32)
        barg_ref[...] = jnp.zeros((S, 1), jnp.int32)
        sumcs_ref[...] = jnp.sum(csize_ref[...], keepdims=True).reshape(1, 1)
        colf_ref[...] = jax.lax.broadcasted_iota(
            jnp.int32, (S, CODE_BLK), 1).astype(jnp.float32)

    mean = mean_ref[...]
    mm2 = jnp.dot(sneg_ref[...], mean, preferred_element_type=jnp.float32)
    m2 = jnp.sum(mean * mean, axis=0, keepdims=True)
    dist = (s2_ref[...] + mm2) + m2

    local_min = jnp.min(dist, axis=1, keepdims=True)
    local_argf = jnp.min(
        jnp.where(dist == local_min, colf_ref[...], jnp.float32(1e9)),
        axis=1, keepdims=True)
    local_arg = local_argf.astype(jnp.int32) + j * CODE_BLK

    better = local_min < best_ref[...]
    best_ref[...] = jnp.where(better, local_min, best_ref[...])
    barg_ref[...] = jnp.where(better, local_arg, barg_ref[...])

    @pl.when(j == N_CODE_BLKS - 1)
    def _done():
        idx_ref[...] = barg_ref[...]


def _stats_body(samples_ref, idx_ref, csum_ref, csize_ref, ns_ref, tab_ref,
                sbf_ref):
    j = pl.program_id(0)

    @pl.when(j == 0)
    def _init():
        sbf_ref[...] = samples_ref[...].astype(jnp.bfloat16)

    idx = idx_ref[...]
    col = jax.lax.broadcasted_iota(jnp.int32, (S, CODE_BLK), 1) + j * CODE_BLK
    onehot = (idx == col).astype(jnp.bfloat16)

    counts = jnp.sum(onehot, axis=0, keepdims=True,
                     dtype=jnp.float32)
    bsum = jax.lax.dot_general(
        sbf_ref[...], onehot, (((0,), (0,)), ((), ())),
        preferred_element_type=jnp.float32)

    csz = csize_ref[:, pl.ds(j * CODE_BLK, CODE_BLK)]
    ns = ns_ref[0, 0]
    usize = csz * DECAY + counts * (1.0 - DECAY)
    sm = (usize + EPS) * ns / (ns + N_EMBED * EPS)
    inv = 1.0 / sm

    nm = (csum_ref[...] * DECAY + bsum * (1.0 - DECAY)) * inv
    tab_ref[...] = nm.T


def _sc_body(samples_h, idx_h, tab_h, out_h, loss_h,
             idxg, gbuf, sbuf, lossv, sem):
    c = lax.axis_index("c")
    s = lax.axis_index("s")
    wid = s * NC + c

    lacc0 = jnp.zeros((16,), jnp.float32)

    for h in range(2):
        base = wid * (2 * HROWS) + h * HROWS
        pltpu.sync_copy(idx_h.at[pl.ds(base, HROWS)], idxg)
        cp = pltpu.async_copy(tab_h.at[idxg], gbuf, sem)
        pltpu.sync_copy(samples_h.at[pl.ds(base, HROWS)], sbuf)
        cp.wait()

        def _row(r, lacc):
            for seg in range(16):
                sl = pl.ds(seg * 16, 16)
                d = sbuf[r, sl] - gbuf[r, sl]
                lacc = lacc + d * d
            return lacc

        lacc0 = lax.fori_loop(0, HROWS, _row, lacc0)
        pltpu.sync_copy(gbuf, out_h.at[pl.ds(base, HROWS)])

    lossv[...] = lacc0
    pltpu.sync_copy(lossv, loss_h.at[pl.ds(wid * 16, 16)])


@jax.jit
def kernel(inputs, cluster_mean, cluster_size, cluster_sum):
    samples = jnp.reshape(inputs, (S, EMBED_DIM))
    csize_2d = jnp.reshape(cluster_size, (1, N_EMBED))

    idx2d, sumcs = pl.pallas_call(
        _argmin_body,
        grid=(N_CODE_BLKS,),
        in_specs=[
            pl.BlockSpec((S, EMBED_DIM), lambda j: (0, 0)),
            pl.BlockSpec((EMBED_DIM, CODE_BLK), lambda j: (0, j)),
            pl.BlockSpec((1, N_EMBED), lambda j: (0, 0)),
        ],
        out_specs=[
            pl.BlockSpec((S, 1), lambda j: (0, 0)),
            pl.BlockSpec((1, 1), lambda j: (0, 0)),
        ],
        out_shape=[
            jax.ShapeDtypeStruct((S, 1), jnp.int32),
            jax.ShapeDtypeStruct((1, 1), jnp.float32),
        ],
        scratch_shapes=[
            pltpu.VMEM((S, 1), jnp.float32),
            pltpu.VMEM((S, 1), jnp.int32),
            pltpu.VMEM((S, 1), jnp.float32),
            pltpu.VMEM((S, EMBED_DIM), jnp.float32),
            pltpu.VMEM((S, CODE_BLK), jnp.float32),
        ],
    )(samples, cluster_mean, csize_2d)

    n_sample = sumcs * DECAY + (1.0 - DECAY) * float(S)

    tab = pl.pallas_call(
        _stats_body,
        grid=(N_CODE_BLKS,),
        in_specs=[
            pl.BlockSpec((S, EMBED_DIM), lambda j: (0, 0)),
            pl.BlockSpec((S, 1), lambda j: (0, 0)),
            pl.BlockSpec((EMBED_DIM, CODE_BLK), lambda j: (0, j)),
            pl.BlockSpec((1, N_EMBED), lambda j: (0, 0)),
            pl.BlockSpec((1, 1), lambda j: (0, 0)),
        ],
        out_specs=pl.BlockSpec((CODE_BLK, EMBED_DIM), lambda j: (j, 0)),
        out_shape=jax.ShapeDtypeStruct((N_EMBED, EMBED_DIM), jnp.float32),
        scratch_shapes=[
            pltpu.VMEM((S, EMBED_DIM), jnp.bfloat16),
        ],
    )(samples, idx2d, cluster_sum, csize_2d, n_sample)

    idx = jnp.reshape(idx2d, (S,))

    sc = pl.kernel(
        _sc_body,
        out_type=[
            jax.ShapeDtypeStruct((S, EMBED_DIM), jnp.float32),
            jax.ShapeDtypeStruct((NC * NS * 16,), jnp.float32),
        ],
        mesh=plsc.VectorSubcoreMesh(core_axis_name="c", subcore_axis_name="s",
                                    num_cores=NC, num_subcores=NS),
        scratch_types=[
            pltpu.VMEM((HROWS,), jnp.int32),
            pltpu.VMEM((HROWS, EMBED_DIM), jnp.float32),
            pltpu.VMEM((HROWS, EMBED_DIM), jnp.float32),
            pltpu.VMEM((16,), jnp.float32),
            pltpu.SemaphoreType.DMA,
        ],
    )
    out, loss_parts = sc(samples, idx, tab)

    outputs = jnp.reshape(out, inputs.shape)
    e_loss = jnp.sum(loss_parts) / float(S * EMBED_DIM)
    return (outputs, 0.25 * e_loss)


# speedup vs baseline: 1.2503x; 1.0623x over previous
import jax
import jax.numpy as jnp
from jax import lax
from jax.experimental import pallas as pl
from jax.experimental.pallas import tpu as pltpu
from jax.experimental.pallas import tpu_sc as plsc

EMBED_DIM = 256
N_EMBED = 8192
DECAY = 0.99
EPS = 1e-05

S = 8 * 576
CODE_BLK = 1024
N_CODE_BLKS = N_EMBED // CODE_BLK

NC, NS = 2, 16
HROWS = S // (NC * NS * 2)


def _tc_body(samples_ref, mean_ref, csize_ref, csum_ref,
             idx_ref, tab_ref,
             best_ref, barg_ref, s2_ref, sneg_ref, sbf_ref, ns_ref):
    j = pl.program_id(0)

    @pl.when(j == 0)
    def _init():
        samples = samples_ref[...]
        sneg_ref[...] = samples * (-2.0)
        sbf_ref[...] = samples.astype(jnp.bfloat16)
        s2_ref[...] = jnp.sum(samples * samples, axis=1, keepdims=True)
        best_ref[...] = jnp.full((S, 1), jnp.inf, jnp.float32)
        barg_ref[...] = jnp.zeros((S, 1), jnp.int32)
        sumcs = jnp.sum(csize_ref[...], keepdims=True).reshape(1, 1)
        ns_ref[...] = sumcs * DECAY + (1.0 - DECAY) * float(S)

    @pl.when(j < N_CODE_BLKS)
    def _argmin_step():
        mean = mean_ref[...]
        mm2 = jnp.dot(sneg_ref[...], mean, preferred_element_type=jnp.float32)
        m2 = jnp.sum(mean * mean, axis=0, keepdims=True)
        dist = (s2_ref[...] + mm2) + m2

        local_min = jnp.min(dist, axis=1, keepdims=True)
        colf = jax.lax.broadcasted_iota(
            jnp.int32, dist.shape, 1).astype(jnp.float32)
        local_argf = jnp.min(
            jnp.where(dist == local_min, colf, jnp.float32(1e9)),
            axis=1, keepdims=True)
        local_arg = local_argf.astype(jnp.int32) + j * CODE_BLK

        better = local_min < best_ref[...]
        best_ref[...] = jnp.where(better, local_min, best_ref[...])
        barg_ref[...] = jnp.where(better, local_arg, barg_ref[...])

        @pl.when(j == N_CODE_BLKS - 1)
        def _done():
            idx_ref[...] = barg_ref[...]

    @pl.when(j >= N_CODE_BLKS)
    def _stats_step():
        jj = j - N_CODE_BLKS
        col = jax.lax.broadcasted_iota(
            jnp.int32, (S, CODE_BLK), 1) + jj * CODE_BLK
        onehot = (barg_ref[...] == col).astype(jnp.bfloat16)

        counts = jnp.sum(onehot, axis=0, keepdims=True,
                         dtype=jnp.float32)
        bsum = jax.lax.dot_general(
            sbf_ref[...], onehot, (((0,), (0,)), ((), ())),
            preferred_element_type=jnp.float32)

        csz = csize_ref[:, pl.ds(jj * CODE_BLK, CODE_BLK)]
        ns = ns_ref[0, 0]
        usize = csz * DECAY + counts * (1.0 - DECAY)
        sm = (usize + EPS) * ns / (ns + N_EMBED * EPS)
        inv = 1.0 / sm

        nm = (csum_ref[...] * DECAY + bsum * (1.0 - DECAY)) * inv
        tab_ref[...] = nm.T


def _sc_body(samples_h, idx_h, tab_h, out_h, loss_h,
             idxg, gbuf, sbuf, lossv, sem):
    c = lax.axis_index("c")
    s = lax.axis_index("s")
    wid = s * NC + c

    lacc0 = jnp.zeros((16,), jnp.float32)

    for h in range(2):
        base = wid * (2 * HROWS) + h * HROWS
        pltpu.sync_copy(idx_h.at[pl.ds(base, HROWS)], idxg)
        cp = pltpu.async_copy(tab_h.at[idxg], gbuf, sem)
        pltpu.sync_copy(samples_h.at[pl.ds(base, HROWS)], sbuf)
        cp.wait()

        def _row(r, lacc):
            for seg in range(16):
                sl = pl.ds(seg * 16, 16)
                d = sbuf[r, sl] - gbuf[r, sl]
                lacc = lacc + d * d
            return lacc

        lacc0 = lax.fori_loop(0, HROWS, _row, lacc0)
        pltpu.sync_copy(gbuf, out_h.at[pl.ds(base, HROWS)])

    lossv[...] = lacc0
    pltpu.sync_copy(lossv, loss_h.at[pl.ds(wid * 16, 16)])


@jax.jit
def kernel(inputs, cluster_mean, cluster_size, cluster_sum):
    samples = jnp.reshape(inputs, (S, EMBED_DIM))
    csize_2d = jnp.reshape(cluster_size, (1, N_EMBED))

    idx2d, tab = pl.pallas_call(
        _tc_body,
        grid=(2 * N_CODE_BLKS,),
        in_specs=[
            pl.BlockSpec((S, EMBED_DIM), lambda j: (0, 0)),
            pl.BlockSpec((EMBED_DIM, CODE_BLK),
                         lambda j: (0, jnp.minimum(j, N_CODE_BLKS - 1))),
            pl.BlockSpec((1, N_EMBED), lambda j: (0, 0)),
            pl.BlockSpec((EMBED_DIM, CODE_BLK),
                         lambda j: (0, jnp.maximum(j - N_CODE_BLKS, 0))),
        ],
        out_specs=[
            pl.BlockSpec((S, 1), lambda j: (0, 0)),
            pl.BlockSpec((CODE_BLK, EMBED_DIM),
                         lambda j: (jnp.maximum(j - N_CODE_BLKS, 0), 0)),
        ],
        out_shape=[
            jax.ShapeDtypeStruct((S, 1), jnp.int32),
            jax.ShapeDtypeStruct((N_EMBED, EMBED_DIM), jnp.float32),
        ],
        scratch_shapes=[
            pltpu.VMEM((S, 1), jnp.float32),
            pltpu.VMEM((S, 1), jnp.int32),
            pltpu.VMEM((S, 1), jnp.float32),
            pltpu.VMEM((S, EMBED_DIM), jnp.float32),
            pltpu.VMEM((S, EMBED_DIM), jnp.bfloat16),
            pltpu.VMEM((1, 1), jnp.float32),
        ],
    )(samples, cluster_mean, csize_2d, cluster_sum)

    idx = jnp.reshape(idx2d, (S,))

    sc = pl.kernel(
        _sc_body,
        out_type=[
            jax.ShapeDtypeStruct((S, EMBED_DIM), jnp.float32),
            jax.ShapeDtypeStruct((NC * NS * 16,), jnp.float32),
        ],
        mesh=plsc.VectorSubcoreMesh(core_axis_name="c", subcore_axis_name="s",
                                    num_cores=NC, num_subcores=NS),
        scratch_types=[
            pltpu.VMEM((HROWS,), jnp.int32),
            pltpu.VMEM((HROWS, EMBED_DIM), jnp.float32),
            pltpu.VMEM((HROWS, EMBED_DIM), jnp.float32),
            pltpu.VMEM((16,), jnp.float32),
            pltpu.SemaphoreType.DMA,
        ],
    )
    out, loss_parts = sc(samples, idx, tab)

    outputs = jnp.reshape(out, inputs.shape)
    e_loss = jnp.sum(loss_parts) / float(S * EMBED_DIM)
    return (outputs, 0.25 * e_loss)


# speedup vs baseline: 1.2682x; 1.0144x over previous
import jax
import jax.numpy as jnp
from jax import lax
from jax.experimental import pallas as pl
from jax.experimental.pallas import tpu as pltpu
from jax.experimental.pallas import tpu_sc as plsc

EMBED_DIM = 256
N_EMBED = 8192
DECAY = 0.99
EPS = 1e-05

S = 8 * 576
CODE_BLK = 1024
N_CODE_BLKS = N_EMBED // CODE_BLK

NC, NS = 2, 16
HROWS = S // (NC * NS * 2)


def _tc_body(samples_ref, mean_ref, csize_ref, csum_ref,
             idx_ref, tab_ref,
             best_ref, barg_ref, s2_ref, sneg_ref, sbf_ref, ns_ref):
    j = pl.program_id(0)

    @pl.when(j == 0)
    def _init():
        samples = samples_ref[...]
        sneg_ref[...] = samples * (-2.0)
        sbf_ref[...] = samples.astype(jnp.bfloat16)
        s2_ref[...] = jnp.sum(samples * samples, axis=1, keepdims=True)
        best_ref[...] = jnp.full((S, 1), jnp.inf, jnp.float32)
        barg_ref[...] = jnp.zeros((S, 1), jnp.int32)
        sumcs = jnp.sum(csize_ref[...], keepdims=True).reshape(1, 1)
        ns_ref[...] = sumcs * DECAY + (1.0 - DECAY) * float(S)

    @pl.when(j < N_CODE_BLKS)
    def _argmin_step():
        mean = mean_ref[...]
        mm2 = jnp.dot(sneg_ref[...], mean, preferred_element_type=jnp.float32)
        m2 = jnp.sum(mean * mean, axis=0, keepdims=True)
        dist = (s2_ref[...] + mm2) + m2

        local_min = jnp.min(dist, axis=1, keepdims=True)
        colf = jax.lax.broadcasted_iota(
            jnp.int32, dist.shape, 1).astype(jnp.float32)
        local_argf = jnp.min(
            jnp.where(dist == local_min, colf, jnp.float32(1e9)),
            axis=1, keepdims=True)
        local_arg = local_argf.astype(jnp.int32) + j * CODE_BLK

        better = local_min < best_ref[...]
        best_ref[...] = jnp.where(better, local_min, best_ref[...])
        barg_ref[...] = jnp.where(better, local_arg, barg_ref[...])

        @pl.when(j == N_CODE_BLKS - 1)
        def _done():
            idx_ref[...] = barg_ref[...]

    @pl.when(j >= N_CODE_BLKS)
    def _stats_step():
        jj = j - N_CODE_BLKS
        col = jax.lax.broadcasted_iota(
            jnp.int32, (S, CODE_BLK), 1) + jj * CODE_BLK
        onehot = (barg_ref[...] == col).astype(jnp.bfloat16)

        counts = jnp.sum(onehot, axis=0, keepdims=True,
                         dtype=jnp.float32)
        bsum = jax.lax.dot_general(
            sbf_ref[...], onehot, (((0,), (0,)), ((), ())),
            preferred_element_type=jnp.float32)

        csz = csize_ref[:, pl.ds(jj * CODE_BLK, CODE_BLK)]
        ns = ns_ref[0, 0]
        usize = csz * DECAY + counts * (1.0 - DECAY)
        sm = (usize + EPS) * ns / (ns + N_EMBED * EPS)
        inv = 1.0 / sm

        nm = (csum_ref[...] * DECAY + bsum * (1.0 - DECAY)) * inv
        tab_ref[...] = nm.T


def _sc_body(samples_h, idx_h, tab_h, out_h, loss_h,
             idxg0, idxg1, gbuf0, gbuf1, sbuf0, sbuf1, lossv,
             semg, sems):
    c = lax.axis_index("c")
    s = lax.axis_index("s")
    wid = s * NC + c
    base0 = wid * (2 * HROWS)
    base1 = base0 + HROWS

    pltpu.sync_copy(idx_h.at[pl.ds(base0, HROWS)], idxg0)
    cg0 = pltpu.async_copy(tab_h.at[idxg0], gbuf0, semg)
    cs0 = pltpu.async_copy(samples_h.at[pl.ds(base0, HROWS)], sbuf0, sems)
    pltpu.sync_copy(idx_h.at[pl.ds(base1, HROWS)], idxg1)
    cg1 = pltpu.async_copy(tab_h.at[idxg1], gbuf1, semg)
    cs1 = pltpu.async_copy(samples_h.at[pl.ds(base1, HROWS)], sbuf1, sems)

    lacc0 = jnp.zeros((16,), jnp.float32)
    for h, (gbuf, sbuf, base, cg, cs) in enumerate(
            ((gbuf0, sbuf0, base0, cg0, cs0), (gbuf1, sbuf1, base1, cg1, cs1))):
        cg.wait()
        cs.wait()

        def _row(r, lacc):
            for seg in range(16):
                sl = pl.ds(seg * 16, 16)
                d = sbuf[r, sl] - gbuf[r, sl]
                lacc = lacc + d * d
            return lacc

        lacc0 = lax.fori_loop(0, HROWS, _row, lacc0)
        pltpu.sync_copy(gbuf, out_h.at[pl.ds(base, HROWS)])

    lossv[...] = lacc0
    pltpu.sync_copy(lossv, loss_h.at[pl.ds(wid * 16, 16)])


@jax.jit
def kernel(inputs, cluster_mean, cluster_size, cluster_sum):
    samples = jnp.reshape(inputs, (S, EMBED_DIM))
    csize_2d = jnp.reshape(cluster_size, (1, N_EMBED))

    idx2d, tab = pl.pallas_call(
        _tc_body,
        grid=(2 * N_CODE_BLKS,),
        in_specs=[
            pl.BlockSpec((S, EMBED_DIM), lambda j: (0, 0)),
            pl.BlockSpec((EMBED_DIM, CODE_BLK),
                         lambda j: (0, jnp.minimum(j, N_CODE_BLKS - 1))),
            pl.BlockSpec((1, N_EMBED), lambda j: (0, 0)),
            pl.BlockSpec((EMBED_DIM, CODE_BLK),
                         lambda j: (0, jnp.maximum(j - N_CODE_BLKS, 0))),
        ],
        out_specs=[
            pl.BlockSpec((S, 1), lambda j: (0, 0)),
            pl.BlockSpec((CODE_BLK, EMBED_DIM),
                         lambda j: (jnp.maximum(j - N_CODE_BLKS, 0), 0)),
        ],
        out_shape=[
            jax.ShapeDtypeStruct((S, 1), jnp.int32),
            jax.ShapeDtypeStruct((N_EMBED, EMBED_DIM), jnp.float32),
        ],
        scratch_shapes=[
            pltpu.VMEM((S, 1), jnp.float32),
            pltpu.VMEM((S, 1), jnp.int32),
            pltpu.VMEM((S, 1), jnp.float32),
            pltpu.VMEM((S, EMBED_DIM), jnp.float32),
            pltpu.VMEM((S, EMBED_DIM), jnp.bfloat16),
            pltpu.VMEM((1, 1), jnp.float32),
        ],
    )(samples, cluster_mean, csize_2d, cluster_sum)

    idx = jnp.reshape(idx2d, (S,))

    sc = pl.kernel(
        _sc_body,
        out_type=[
            jax.ShapeDtypeStruct((S, EMBED_DIM), jnp.float32),
            jax.ShapeDtypeStruct((NC * NS * 16,), jnp.float32),
        ],
        mesh=plsc.VectorSubcoreMesh(core_axis_name="c", subcore_axis_name="s",
                                    num_cores=NC, num_subcores=NS),
        scratch_types=[
            pltpu.VMEM((HROWS,), jnp.int32),
            pltpu.VMEM((HROWS,), jnp.int32),
            pltpu.VMEM((HROWS, EMBED_DIM), jnp.float32),
            pltpu.VMEM((HROWS, EMBED_DIM), jnp.float32),
            pltpu.VMEM((HROWS, EMBED_DIM), jnp.float32),
            pltpu.VMEM((HROWS, EMBED_DIM), jnp.float32),
            pltpu.VMEM((16,), jnp.float32),
            pltpu.SemaphoreType.DMA,
            pltpu.SemaphoreType.DMA,
        ],
    )
    out, loss_parts = sc(samples, idx, tab)

    outputs = jnp.reshape(out, inputs.shape)
    e_loss = jnp.sum(loss_parts) / float(S * EMBED_DIM)
    return (outputs, 0.25 * e_loss)
